# baseline (speedup 1.0000x reference)
import functools

import jax
import jax.numpy as jnp
from jax import lax
from jax.experimental import pallas as pl
from jax.experimental.pallas import tpu as pltpu
from jax.experimental.pallas import tpu_sc as plsc

HIDDEN = 128
LANES = 16
NH = HIDDEN // LANES
EPS = 1e-12


def _rsqrt(x):
    i = plsc.bitcast(x, jnp.int32)
    i = 0x5F3759DF - lax.shift_right_logical(i, 1)
    y = plsc.bitcast(i, jnp.float32)
    for _ in range(3):
        y = y * (1.5 - 0.5 * x * y * y)
    return y


def _body(n_per_w, nc, ids_hbm, tti_hbm, wtab_hbm, tttab_hbm, gam_hbm, bet_hbm,
          out_hbm, idx_v, tti_v, rows_v, ttrows_v, gam_v, bet_v, sem):
    wid = lax.axis_index("s") * nc + lax.axis_index("c")
    base = wid * n_per_w

    pltpu.sync_copy(ids_hbm.at[pl.ds(base, n_per_w)], idx_v)
    pltpu.sync_copy(tti_hbm.at[pl.ds(base, n_per_w)], tti_v)
    pltpu.sync_copy(gam_hbm, gam_v)
    pltpu.sync_copy(bet_hbm, bet_v)

    nchunk = n_per_w // 128
    copies = []
    for c in range(nchunk):
        sl = pl.ds(c * 128, 128)
        copies.append(pltpu.async_copy(wtab_hbm.at[idx_v.at[sl]], rows_v.at[sl], sem))
        copies.append(pltpu.async_copy(tttab_hbm.at[tti_v.at[sl]], ttrows_v.at[sl], sem))
    for cp in copies:
        cp.wait()

    gam = [gam_v[pl.ds(h * LANES, LANES)] for h in range(NH)]
    bet = [bet_v[pl.ds(h * LANES, LANES)] for h in range(NH)]

    def token(t, carry):
        e = [rows_v[t, pl.ds(h * LANES, LANES)] + ttrows_v[t, pl.ds(h * LANES, LANES)]
             for h in range(NH)]
        s = e[0]
        for h in range(1, NH):
            s = s + e[h]
        mean = jnp.sum(s) * (1.0 / HIDDEN)
        d = [eh - mean for eh in e]
        sq = d[0] * d[0]
        for h in range(1, NH):
            sq = sq + d[h] * d[h]
        var = jnp.sum(sq) * (1.0 / HIDDEN)
        rinv = _rsqrt(jnp.full((LANES,), var + EPS, jnp.float32))
        for h in range(NH):
            rows_v[t, pl.ds(h * LANES, LANES)] = d[h] * rinv * gam[h] + bet[h]
        return carry

    lax.fori_loop(0, n_per_w, token, 0, unroll=False)

    pltpu.sync_copy(rows_v, out_hbm.at[pl.ds(base, n_per_w)])


def kernel(input_ids, token_type_ids, word_embeddings, token_type_embeddings,
           ln_gamma, ln_beta):
    b, s = input_ids.shape
    n = b * s
    nc, ns = 2, 16
    nw = nc * ns
    n_per_w = n // nw

    mesh = plsc.VectorSubcoreMesh(core_axis_name="c", subcore_axis_name="s")
    run = pl.kernel(
        functools.partial(_body, n_per_w, nc),
        out_type=jax.ShapeDtypeStruct((n, HIDDEN), jnp.float32),
        mesh=mesh,
        scratch_types=[
            pltpu.VMEM((n_per_w,), jnp.int32),
            pltpu.VMEM((n_per_w,), jnp.int32),
            pltpu.VMEM((n_per_w, HIDDEN), jnp.float32),
            pltpu.VMEM((n_per_w, HIDDEN), jnp.float32),
            pltpu.VMEM((HIDDEN,), jnp.float32),
            pltpu.VMEM((HIDDEN,), jnp.float32),
            pltpu.SemaphoreType.DMA,
        ],
        compiler_params=pltpu.CompilerParams(needs_layout_passes=False),
    )
    out = run(
        input_ids.reshape(n).astype(jnp.int32),
        token_type_ids.reshape(n).astype(jnp.int32),
        word_embeddings,
        token_type_embeddings,
        ln_gamma,
        ln_beta,
    )
    return out.reshape(b, s, HIDDEN)
